# Initial kernel scaffold; baseline (speedup 1.0000x reference)
import jax
import jax.numpy as jnp
from jax.experimental import pallas as pl


def kernel(inputs, centroids):
    raise NotImplementedError("write your pallas kernel here")


# speedup vs baseline: 34.8895x; 34.8895x over previous
import jax
import jax.numpy as jnp
from jax.experimental import pallas as pl
from jax.experimental.pallas import tpu as pltpu

_BN = 512
_BK = 256


def _tile_kernel(x_ref, c_ref, dist_ref, assign_ref, best_val, best_idx):
    k = pl.program_id(1)
    nk = pl.num_programs(1)

    x = x_ref[...]
    c = c_ref[...]
    x2 = jnp.sum(x * x, axis=1)[None, :]
    c2 = jnp.sum(c * c, axis=1)[:, None]
    dots = jax.lax.dot_general(
        c, x, (((1,), (1,)), ((), ())),
        preferred_element_type=jnp.float32,
        precision=jax.lax.Precision.HIGHEST)
    dist = (c2 - 2.0 * dots) + x2
    dist_ref[...] = dist

    local_min = jnp.min(dist, axis=0, keepdims=True)
    rows = jax.lax.broadcasted_iota(jnp.int32, dist.shape, 0)
    big = jnp.int32(jnp.iinfo(jnp.int32).max)
    local_arg = jnp.min(jnp.where(dist == local_min, rows, big),
                        axis=0, keepdims=True) + k * dist.shape[0]

    @pl.when(k == 0)
    def _():
        best_val[...] = local_min
        best_idx[...] = local_arg

    @pl.when(k > 0)
    def _():
        bv = best_val[...]
        take = local_min < bv
        best_val[...] = jnp.where(take, local_min, bv)
        best_idx[...] = jnp.where(take, local_arg, best_idx[...])

    @pl.when(k == nk - 1)
    def _():
        assign_ref[...] = best_idx[...]


def kernel(inputs, centroids):
    n, d = inputs.shape
    k, _ = centroids.shape
    bn, bk = _BN, _BK
    grid = (n // bn, k // bk)
    dist, assign = pl.pallas_call(
        _tile_kernel,
        grid=grid,
        in_specs=[
            pl.BlockSpec((bn, d), lambda j, i: (j, 0)),
            pl.BlockSpec((bk, d), lambda j, i: (i, 0)),
        ],
        out_specs=[
            pl.BlockSpec((bk, bn), lambda j, i: (i, j)),
            pl.BlockSpec((1, bn), lambda j, i: (0, j)),
        ],
        out_shape=[
            jax.ShapeDtypeStruct((k, n), jnp.float32),
            jax.ShapeDtypeStruct((1, n), jnp.int32),
        ],
        scratch_shapes=[
            pltpu.VMEM((1, bn), jnp.float32),
            pltpu.VMEM((1, bn), jnp.int32),
        ],
        compiler_params=pltpu.CompilerParams(
            dimension_semantics=("parallel", "arbitrary")),
    )(inputs, centroids)
    return dist, assign[0]


# speedup vs baseline: 46.6882x; 1.3382x over previous
import jax
import jax.numpy as jnp
from jax.experimental import pallas as pl
from jax.experimental.pallas import tpu as pltpu

_BN = 1024
_BK = 256


def _tile_kernel(x_ref, c_ref, dist_ref, assign_ref, best_val, best_idx):
    k = pl.program_id(1)
    nk = pl.num_programs(1)

    x = x_ref[...]
    c = c_ref[...]
    x2 = jnp.sum(x * x, axis=1)[None, :]
    c2 = jnp.sum(c * c, axis=1)[:, None]
    dots = jax.lax.dot_general(
        c, x, (((1,), (1,)), ((), ())),
        preferred_element_type=jnp.float32,
        precision=jax.lax.Precision.HIGHEST)
    dist = (c2 - 2.0 * dots) + x2
    dist_ref[...] = dist

    local_min = jnp.min(dist, axis=0, keepdims=True)
    rows = jax.lax.broadcasted_iota(jnp.int32, dist.shape, 0)
    big = jnp.int32(jnp.iinfo(jnp.int32).max)
    local_arg = jnp.min(jnp.where(dist == local_min, rows, big),
                        axis=0, keepdims=True) + k * dist.shape[0]

    @pl.when(k == 0)
    def _():
        best_val[...] = local_min
        best_idx[...] = local_arg

    @pl.when(k > 0)
    def _():
        bv = best_val[...]
        take = local_min < bv
        best_val[...] = jnp.where(take, local_min, bv)
        best_idx[...] = jnp.where(take, local_arg, best_idx[...])

    @pl.when(k == nk - 1)
    def _():
        assign_ref[...] = best_idx[...]


def kernel(inputs, centroids):
    n, d = inputs.shape
    k, _ = centroids.shape
    bn, bk = _BN, _BK
    grid = (n // bn, k // bk)
    dist, assign = pl.pallas_call(
        _tile_kernel,
        grid=grid,
        in_specs=[
            pl.BlockSpec((bn, d), lambda j, i: (j, 0)),
            pl.BlockSpec((bk, d), lambda j, i: (i, 0)),
        ],
        out_specs=[
            pl.BlockSpec((bk, bn), lambda j, i: (i, j)),
            pl.BlockSpec((1, bn), lambda j, i: (0, j)),
        ],
        out_shape=[
            jax.ShapeDtypeStruct((k, n), jnp.float32),
            jax.ShapeDtypeStruct((1, n), jnp.int32),
        ],
        scratch_shapes=[
            pltpu.VMEM((1, bn), jnp.float32),
            pltpu.VMEM((1, bn), jnp.int32),
        ],
        compiler_params=pltpu.CompilerParams(
            dimension_semantics=("parallel", "arbitrary")),
    )(inputs, centroids)
    return dist, assign[0]


# speedup vs baseline: 54.0785x; 1.1583x over previous
import jax
import jax.numpy as jnp
from jax.experimental import pallas as pl
from jax.experimental.pallas import tpu as pltpu

_BN = 2048
_BK = 256


def _tile_kernel(x_ref, c_ref, dist_ref, assign_ref, best_val, best_idx):
    k = pl.program_id(1)
    nk = pl.num_programs(1)

    x = x_ref[...]
    c = c_ref[...]
    x2 = jnp.sum(x * x, axis=1)[None, :]
    c2 = jnp.sum(c * c, axis=1)[:, None]
    dots = jax.lax.dot_general(
        c, x, (((1,), (1,)), ((), ())),
        preferred_element_type=jnp.float32,
        precision=jax.lax.Precision.HIGHEST)
    dist = (c2 - 2.0 * dots) + x2
    dist_ref[...] = dist

    local_min = jnp.min(dist, axis=0, keepdims=True)
    rows = jax.lax.broadcasted_iota(jnp.int32, dist.shape, 0)
    big = jnp.int32(jnp.iinfo(jnp.int32).max)
    local_arg = jnp.min(jnp.where(dist == local_min, rows, big),
                        axis=0, keepdims=True) + k * dist.shape[0]

    @pl.when(k == 0)
    def _():
        best_val[...] = local_min
        best_idx[...] = local_arg

    @pl.when(k > 0)
    def _():
        bv = best_val[...]
        take = local_min < bv
        best_val[...] = jnp.where(take, local_min, bv)
        best_idx[...] = jnp.where(take, local_arg, best_idx[...])

    @pl.when(k == nk - 1)
    def _():
        assign_ref[...] = best_idx[...]


def kernel(inputs, centroids):
    n, d = inputs.shape
    k, _ = centroids.shape
    bn, bk = _BN, _BK
    grid = (n // bn, k // bk)
    dist, assign = pl.pallas_call(
        _tile_kernel,
        grid=grid,
        in_specs=[
            pl.BlockSpec((bn, d), lambda j, i: (j, 0)),
            pl.BlockSpec((bk, d), lambda j, i: (i, 0)),
        ],
        out_specs=[
            pl.BlockSpec((bk, bn), lambda j, i: (i, j)),
            pl.BlockSpec((1, bn), lambda j, i: (0, j)),
        ],
        out_shape=[
            jax.ShapeDtypeStruct((k, n), jnp.float32),
            jax.ShapeDtypeStruct((1, n), jnp.int32),
        ],
        scratch_shapes=[
            pltpu.VMEM((1, bn), jnp.float32),
            pltpu.VMEM((1, bn), jnp.int32),
        ],
        compiler_params=pltpu.CompilerParams(
            dimension_semantics=("parallel", "arbitrary")),
    )(inputs, centroids)
    return dist, assign[0]


# speedup vs baseline: 55.0699x; 1.0183x over previous
import jax
import jax.numpy as jnp
from jax.experimental import pallas as pl
from jax.experimental.pallas import tpu as pltpu

_BN = 4096
_BK = 256


def _tile_kernel(x_ref, c_ref, dist_ref, assign_ref, best_val, best_idx):
    k = pl.program_id(1)
    nk = pl.num_programs(1)

    x = x_ref[...]
    c = c_ref[...]
    x2 = jnp.sum(x * x, axis=1)[None, :]
    c2 = jnp.sum(c * c, axis=1)[:, None]
    dots = jax.lax.dot_general(
        c, x, (((1,), (1,)), ((), ())),
        preferred_element_type=jnp.float32,
        precision=jax.lax.Precision.HIGHEST)
    dist = (c2 - 2.0 * dots) + x2
    dist_ref[...] = dist

    local_min = jnp.min(dist, axis=0, keepdims=True)
    rows = jax.lax.broadcasted_iota(jnp.int32, dist.shape, 0)
    big = jnp.int32(jnp.iinfo(jnp.int32).max)
    local_arg = jnp.min(jnp.where(dist == local_min, rows, big),
                        axis=0, keepdims=True) + k * dist.shape[0]

    @pl.when(k == 0)
    def _():
        best_val[...] = local_min
        best_idx[...] = local_arg

    @pl.when(k > 0)
    def _():
        bv = best_val[...]
        take = local_min < bv
        best_val[...] = jnp.where(take, local_min, bv)
        best_idx[...] = jnp.where(take, local_arg, best_idx[...])

    @pl.when(k == nk - 1)
    def _():
        assign_ref[...] = best_idx[...]


def kernel(inputs, centroids):
    n, d = inputs.shape
    k, _ = centroids.shape
    bn, bk = _BN, _BK
    grid = (n // bn, k // bk)
    dist, assign = pl.pallas_call(
        _tile_kernel,
        grid=grid,
        in_specs=[
            pl.BlockSpec((bn, d), lambda j, i: (j, 0)),
            pl.BlockSpec((bk, d), lambda j, i: (i, 0)),
        ],
        out_specs=[
            pl.BlockSpec((bk, bn), lambda j, i: (i, j)),
            pl.BlockSpec((1, bn), lambda j, i: (0, j)),
        ],
        out_shape=[
            jax.ShapeDtypeStruct((k, n), jnp.float32),
            jax.ShapeDtypeStruct((1, n), jnp.int32),
        ],
        scratch_shapes=[
            pltpu.VMEM((1, bn), jnp.float32),
            pltpu.VMEM((1, bn), jnp.int32),
        ],
        compiler_params=pltpu.CompilerParams(
            dimension_semantics=("parallel", "arbitrary")),
    )(inputs, centroids)
    return dist, assign[0]
